# baseline (speedup 1.0000x reference)
import functools

import jax
import jax.numpy as jnp
from jax import lax
from jax.experimental import pallas as pl
from jax.experimental.pallas import tpu as pltpu
from jax.experimental.pallas import tpu_sc as plsc

N = 10000
E = 320000
D = 128
EPS = 1e-5

NC = 2
NS = 16
NW = NC * NS
EPW = E // NW
CH = 80
NCH = EPW // CH
RPT = 632
NP = NS * RPT

_sc_mesh = plsc.VectorSubcoreMesh(core_axis_name="c", subcore_axis_name="s")

NB1 = 3
NB3 = 4


@functools.partial(
    pl.kernel,
    mesh=_sc_mesh,
    out_type=jax.ShapeDtypeStruct((NC, NP, D), jnp.float32),
    scratch_types=[
        [pltpu.VMEM((CH, D), jnp.float32) for _ in range(NB1)],
        pltpu.VMEM((NCH, CH), jnp.int32),
        pltpu.VMEM_SHARED((NP, D), jnp.float32),
        [pltpu.SemaphoreType.DMA for _ in range(NB1)],
    ],
)
def _seg_sum_sc(ef_hbm, recv_hbm, zeros_hbm, out_hbm, ebufs, ibuf, acc, sems):
    c = lax.axis_index("c")
    s = lax.axis_index("s")
    wid = s * NC + c
    pltpu.sync_copy(zeros_hbm, acc.at[pl.ds(s * RPT, RPT)])
    pltpu.sync_copy(recv_hbm.at[wid], ibuf)
    plsc.subcore_barrier()

    for b in range(NB1):
        pltpu.async_copy(ef_hbm.at[pl.ds(wid * EPW + b * CH, CH)], ebufs[b], sems[b])

    def step(k, b):
        base = wid * EPW + k * CH
        pltpu.make_async_copy(ef_hbm.at[pl.ds(base, CH)], ebufs[b], sems[b]).wait()
        pltpu.sync_copy(ebufs[b], acc.at[ibuf.at[k]], add=True)

    def group(gg, carry):
        g = gg * NB1
        for b in range(NB1):
            k = g + b
            step(k, b)

            @pl.when(k + NB1 < NCH)
            def _():
                pltpu.async_copy(
                    ef_hbm.at[pl.ds(wid * EPW + (k + NB1) * CH, CH)],
                    ebufs[b], sems[b])
        return carry

    lax.fori_loop(0, NCH // NB1, group, 0)
    for k in range((NCH // NB1) * NB1, NCH):
        step(k, k % NB1)
    plsc.subcore_barrier()
    pltpu.sync_copy(acc.at[pl.ds(s * RPT, RPT)], out_hbm.at[c, pl.ds(s * RPT, RPT)])


@functools.partial(
    pl.kernel,
    mesh=_sc_mesh,
    out_type=jax.ShapeDtypeStruct((E, D), jnp.float32),
    scratch_types=[
        [pltpu.VMEM((CH, D), jnp.float32) for _ in range(NB3)],
        [pltpu.VMEM((CH, D), jnp.float32) for _ in range(NB3)],
        pltpu.VMEM((NCH, CH), jnp.int32),
        pltpu.VMEM((NCH, CH), jnp.int32),
        [pltpu.SemaphoreType.DMA for _ in range(NB3)],
        [pltpu.SemaphoreType.DMA for _ in range(NB3)],
    ],
)
def _gather_sc(p_hbm, q_hbm, snd_hbm, rcv_hbm, out_hbm,
               pbufs, qbufs, sbuf, rbuf, psems, qsems):
    c = lax.axis_index("c")
    s = lax.axis_index("s")
    wid = s * NC + c
    pltpu.sync_copy(snd_hbm.at[wid], sbuf)
    pltpu.sync_copy(rcv_hbm.at[wid], rbuf)

    for b in range(NB3):
        pltpu.async_copy(p_hbm.at[sbuf.at[b]], pbufs[b], psems[b])
        pltpu.async_copy(q_hbm.at[rbuf.at[b]], qbufs[b], qsems[b])

    def step(k, b):
        base = wid * EPW + k * CH
        pltpu.make_async_copy(p_hbm.at[sbuf.at[k]], pbufs[b], psems[b]).wait()
        pltpu.make_async_copy(q_hbm.at[rbuf.at[k]], qbufs[b], qsems[b]).wait()

        def add_row(i, cc):
            for j in range(D // 16):
                sl = pl.ds(j * 16, 16)
                pbufs[b][i, sl] = pbufs[b][i, sl] + qbufs[b][i, sl]
            return cc

        lax.fori_loop(0, CH, add_row, 0)
        pltpu.sync_copy(pbufs[b], out_hbm.at[pl.ds(base, CH)])

    def group(gg, carry):
        g = gg * NB3
        for b in range(NB3):
            k = g + b
            step(k, b)

            @pl.when(k + NB3 < NCH)
            def _():
                pltpu.async_copy(p_hbm.at[sbuf.at[k + NB3]], pbufs[b], psems[b])
                pltpu.async_copy(q_hbm.at[rbuf.at[k + NB3]], qbufs[b], qsems[b])
        return carry

    lax.fori_loop(0, NCH // NB3, group, 0)
    for k in range((NCH // NB3) * NB3, NCH):
        step(k, k % NB3)


def _node_mlp_body(nf_ref, agg_ref, Wn1_ref, bn1_ref, Wn2_ref, bn2_ref,
                   gn_ref, btn_ref, We1_ref, We2_ref, nodes_ref, p_ref, q_ref):
    agg = agg_ref[0] + agg_ref[1]
    h = jnp.dot(nf_ref[...], Wn1_ref[:D], preferred_element_type=jnp.float32)
    h = h + jnp.dot(agg, Wn1_ref[D:], preferred_element_type=jnp.float32)
    h = h + bn1_ref[...]
    x = jnp.dot(h, Wn2_ref[...], preferred_element_type=jnp.float32) + bn2_ref[...]
    mu = jnp.mean(x, axis=-1, keepdims=True)
    dd = x - mu
    var = jnp.mean(dd * dd, axis=-1, keepdims=True)
    nodes = (dd * lax.rsqrt(var + EPS)) * gn_ref[...] + btn_ref[...]
    nodes_ref[...] = nodes
    ws2 = jnp.dot(We1_ref[:D], We2_ref[...], preferred_element_type=jnp.float32)
    wr2 = jnp.dot(We1_ref[D:2 * D], We2_ref[...], preferred_element_type=jnp.float32)

    p_ref[...] = jnp.dot(nodes, ws2, preferred_element_type=jnp.float32)
    q_ref[...] = jnp.dot(nodes, wr2, preferred_element_type=jnp.float32)


def _edge_mlp_body(g_ref, ef_ref, We1_ref, be1_ref, We2_ref, be2_ref,
                   ge_ref, bte_ref, out_ref):
    we2 = jnp.dot(We1_ref[2 * D:], We2_ref[...], preferred_element_type=jnp.float32)
    b2 = jnp.dot(be1_ref[...], We2_ref[...], preferred_element_type=jnp.float32) \
        + be2_ref[...]
    ef16 = ef_ref[...].astype(jnp.bfloat16)
    we216 = we2.astype(jnp.bfloat16)
    x = g_ref[...] + jnp.dot(ef16, we216, preferred_element_type=jnp.float32) + b2
    mu = jnp.mean(x, axis=-1, keepdims=True)
    dd = x - mu
    var = jnp.mean(dd * dd, axis=-1, keepdims=True)
    out_ref[...] = (dd * lax.rsqrt(var + EPS)) * ge_ref[...] + bte_ref[...]


BN = 2000
BE = 2560


def _full(shape):
    return pl.BlockSpec(shape, lambda i: (0,) * len(shape))


def kernel(node_features, edge_features, senders, receivers,
           Wn1, bn1, Wn2, bn2, gn, btn, We1, be1, We2, be2, ge, bte):
    zeros = jnp.zeros((RPT, D), jnp.float32)
    snd3 = senders.reshape(NW, NCH, CH)
    rcv3 = receivers.reshape(NW, NCH, CH)

    bn1r, bn2r = bn1.reshape(1, D), bn2.reshape(1, D)
    gnr, btnr = gn.reshape(1, D), btn.reshape(1, D)
    be1r, be2r = be1.reshape(1, D), be2.reshape(1, D)
    ger, bter = ge.reshape(1, D), bte.reshape(1, D)

    partials = _seg_sum_sc(edge_features, rcv3, zeros)

    rowspec = pl.BlockSpec((BN, D), lambda i: (i, 0))
    new_nodes, p, q = pl.pallas_call(
        _node_mlp_body,
        grid=(N // BN,),
        in_specs=[
            rowspec,
            pl.BlockSpec((NC, BN, D), lambda i: (0, i, 0)),
            _full((2 * D, D)), _full((1, D)),
            _full((D, D)), _full((1, D)),
            _full((1, D)), _full((1, D)),
            _full((3 * D, D)), _full((D, D)),
        ],
        out_specs=[rowspec, rowspec, rowspec],
        out_shape=[jax.ShapeDtypeStruct((N, D), jnp.float32)] * 3,
    )(node_features, partials, Wn1, bn1r, Wn2, bn2r, gnr, btnr, We1, We2)

    g = _gather_sc(p, q, snd3, rcv3)

    erowspec = pl.BlockSpec((BE, D), lambda i: (i, 0))
    new_edges = pl.pallas_call(
        _edge_mlp_body,
        grid=(E // BE,),
        in_specs=[
            erowspec, erowspec,
            _full((3 * D, D)), _full((1, D)),
            _full((D, D)), _full((1, D)),
            _full((1, D)), _full((1, D)),
        ],
        out_specs=erowspec,
        out_shape=jax.ShapeDtypeStruct((E, D), jnp.float32),
    )(g, edge_features, We1, be1r, We2, be2r, ger, bter)

    return (new_nodes, new_edges)


# speedup vs baseline: 1.0989x; 1.0989x over previous
import functools

import jax
import jax.numpy as jnp
from jax import lax
from jax.experimental import pallas as pl
from jax.experimental.pallas import tpu as pltpu
from jax.experimental.pallas import tpu_sc as plsc

N = 10000
E = 320000
D = 128
EPS = 1e-5

NC = 2
NS = 16
NW = NC * NS
EPW = E // NW
CH = 80
NCH = EPW // CH
RPT = 632
NP = NS * RPT

_sc_mesh = plsc.VectorSubcoreMesh(core_axis_name="c", subcore_axis_name="s")

NB1 = 3
NB3 = 4


@functools.partial(
    pl.kernel,
    mesh=_sc_mesh,
    out_type=jax.ShapeDtypeStruct((NC, NP, D), jnp.float32),
    scratch_types=[
        [pltpu.VMEM((CH, D), jnp.float32) for _ in range(NB1)],
        pltpu.VMEM((NCH, CH), jnp.int32),
        pltpu.VMEM_SHARED((NP, D), jnp.float32),
        [pltpu.SemaphoreType.DMA for _ in range(NB1)],
    ],
)
def _seg_sum_sc(ef_hbm, recv_hbm, zeros_hbm, out_hbm, ebufs, ibuf, acc, sems):
    c = lax.axis_index("c")
    s = lax.axis_index("s")
    wid = s * NC + c
    pltpu.sync_copy(zeros_hbm, acc.at[pl.ds(s * RPT, RPT)])
    pltpu.sync_copy(recv_hbm.at[wid], ibuf)
    plsc.subcore_barrier()

    for b in range(NB1):
        pltpu.async_copy(ef_hbm.at[pl.ds(wid * EPW + b * CH, CH)], ebufs[b], sems[b])

    def step(k, b):
        base = wid * EPW + k * CH
        pltpu.make_async_copy(ef_hbm.at[pl.ds(base, CH)], ebufs[b], sems[b]).wait()
        pltpu.sync_copy(ebufs[b], acc.at[ibuf.at[k]], add=True)

    def group(gg, carry):
        g = gg * NB1
        for b in range(NB1):
            k = g + b
            step(k, b)

            @pl.when(k + NB1 < NCH)
            def _():
                pltpu.async_copy(
                    ef_hbm.at[pl.ds(wid * EPW + (k + NB1) * CH, CH)],
                    ebufs[b], sems[b])
        return carry

    lax.fori_loop(0, NCH // NB1, group, 0)
    for k in range((NCH // NB1) * NB1, NCH):
        step(k, k % NB1)
    plsc.subcore_barrier()
    pltpu.sync_copy(acc.at[pl.ds(s * RPT, RPT)], out_hbm.at[c, pl.ds(s * RPT, RPT)])


@functools.partial(
    pl.kernel,
    mesh=_sc_mesh,
    out_type=jax.ShapeDtypeStruct((E, D), jnp.float32),
    scratch_types=[
        [pltpu.VMEM((CH, D), jnp.float32) for _ in range(NB3)],
        [pltpu.VMEM((CH, D), jnp.float32) for _ in range(NB3)],
        pltpu.VMEM((NCH, CH), jnp.int32),
        pltpu.VMEM((NCH, CH), jnp.int32),
        [pltpu.SemaphoreType.DMA for _ in range(NB3)],
        [pltpu.SemaphoreType.DMA for _ in range(NB3)],
    ],
)
def _gather_sc(p_hbm, q_hbm, snd_hbm, rcv_hbm, out_hbm,
               pbufs, qbufs, sbuf, rbuf, psems, qsems):
    c = lax.axis_index("c")
    s = lax.axis_index("s")
    wid = s * NC + c
    pltpu.sync_copy(snd_hbm.at[wid], sbuf)
    pltpu.sync_copy(rcv_hbm.at[wid], rbuf)

    for b in range(NB3):
        pltpu.async_copy(p_hbm.at[sbuf.at[b]], pbufs[b], psems[b])
        pltpu.async_copy(q_hbm.at[rbuf.at[b]], qbufs[b], qsems[b])

    def step(k, b):
        base = wid * EPW + k * CH
        pltpu.make_async_copy(p_hbm.at[sbuf.at[k]], pbufs[b], psems[b]).wait()
        pltpu.make_async_copy(q_hbm.at[rbuf.at[k]], qbufs[b], qsems[b]).wait()

        def add_row(i, cc):
            for j in range(D // 16):
                sl = pl.ds(j * 16, 16)
                pbufs[b][i, sl] = pbufs[b][i, sl] + qbufs[b][i, sl]
            return cc

        lax.fori_loop(0, CH, add_row, 0)
        pltpu.sync_copy(pbufs[b], out_hbm.at[pl.ds(base, CH)])

    def group(gg, carry):
        g = gg * NB3
        for b in range(NB3):
            k = g + b
            step(k, b)

            @pl.when(k + NB3 < NCH)
            def _():
                pltpu.async_copy(p_hbm.at[sbuf.at[k + NB3]], pbufs[b], psems[b])
                pltpu.async_copy(q_hbm.at[rbuf.at[k + NB3]], qbufs[b], qsems[b])
        return carry

    lax.fori_loop(0, NCH // NB3, group, 0)
    for k in range((NCH // NB3) * NB3, NCH):
        step(k, k % NB3)


def _node_mlp_body(nf_ref, agg_ref, Wn1_ref, bn1_ref, Wn2_ref, bn2_ref,
                   gn_ref, btn_ref, We1_ref, We2_ref, nodes_ref, p_ref, q_ref):
    agg = agg_ref[0] + agg_ref[1]
    h = jnp.dot(nf_ref[...], Wn1_ref[:D], preferred_element_type=jnp.float32)
    h = h + jnp.dot(agg, Wn1_ref[D:], preferred_element_type=jnp.float32)
    h = h + bn1_ref[...]
    x = jnp.dot(h, Wn2_ref[...], preferred_element_type=jnp.float32) + bn2_ref[...]
    mu = jnp.mean(x, axis=-1, keepdims=True)
    dd = x - mu
    var = jnp.mean(dd * dd, axis=-1, keepdims=True)
    nodes = (dd * lax.rsqrt(var + EPS)) * gn_ref[...] + btn_ref[...]
    nodes_ref[...] = nodes
    ws2 = jnp.dot(We1_ref[:D], We2_ref[...], preferred_element_type=jnp.float32)
    wr2 = jnp.dot(We1_ref[D:2 * D], We2_ref[...], preferred_element_type=jnp.float32)

    p_ref[...] = jnp.dot(nodes, ws2, preferred_element_type=jnp.float32)
    q_ref[...] = jnp.dot(nodes, wr2, preferred_element_type=jnp.float32)


def _edge_mlp_body(g_ref, ef_ref, We1_ref, be1_ref, We2_ref, be2_ref,
                   ge_ref, bte_ref, out_ref):
    we2 = jnp.dot(We1_ref[2 * D:], We2_ref[...], preferred_element_type=jnp.float32)
    b2 = jnp.dot(be1_ref[...], We2_ref[...], preferred_element_type=jnp.float32) \
        + be2_ref[...]
    ef16 = ef_ref[...].astype(jnp.bfloat16)
    we216 = we2.astype(jnp.bfloat16)
    x = g_ref[...] + jnp.dot(ef16, we216, preferred_element_type=jnp.float32) + b2
    mu = jnp.mean(x, axis=-1, keepdims=True)
    dd = x - mu
    var = jnp.mean(dd * dd, axis=-1, keepdims=True)
    out_ref[...] = (dd * lax.rsqrt(var + EPS)) * ge_ref[...] + bte_ref[...]


BN = 2000
BE = 5000


def _full(shape):
    return pl.BlockSpec(shape, lambda i: (0,) * len(shape))


def kernel(node_features, edge_features, senders, receivers,
           Wn1, bn1, Wn2, bn2, gn, btn, We1, be1, We2, be2, ge, bte):
    zeros = jnp.zeros((RPT, D), jnp.float32)
    snd3 = senders.reshape(NW, NCH, CH)
    rcv3 = receivers.reshape(NW, NCH, CH)

    bn1r, bn2r = bn1.reshape(1, D), bn2.reshape(1, D)
    gnr, btnr = gn.reshape(1, D), btn.reshape(1, D)
    be1r, be2r = be1.reshape(1, D), be2.reshape(1, D)
    ger, bter = ge.reshape(1, D), bte.reshape(1, D)

    partials = _seg_sum_sc(edge_features, rcv3, zeros)

    rowspec = pl.BlockSpec((BN, D), lambda i: (i, 0))
    new_nodes, p, q = pl.pallas_call(
        _node_mlp_body,
        grid=(N // BN,),
        in_specs=[
            rowspec,
            pl.BlockSpec((NC, BN, D), lambda i: (0, i, 0)),
            _full((2 * D, D)), _full((1, D)),
            _full((D, D)), _full((1, D)),
            _full((1, D)), _full((1, D)),
            _full((3 * D, D)), _full((D, D)),
        ],
        out_specs=[rowspec, rowspec, rowspec],
        out_shape=[jax.ShapeDtypeStruct((N, D), jnp.float32)] * 3,
    )(node_features, partials, Wn1, bn1r, Wn2, bn2r, gnr, btnr, We1, We2)

    g = _gather_sc(p, q, snd3, rcv3)

    erowspec = pl.BlockSpec((BE, D), lambda i: (i, 0))
    new_edges = pl.pallas_call(
        _edge_mlp_body,
        grid=(E // BE,),
        in_specs=[
            erowspec, erowspec,
            _full((3 * D, D)), _full((1, D)),
            _full((D, D)), _full((1, D)),
            _full((1, D)), _full((1, D)),
        ],
        out_specs=erowspec,
        out_shape=jax.ShapeDtypeStruct((E, D), jnp.float32),
    )(g, edge_features, We1, be1r, We2, be2r, ger, bter)

    return (new_nodes, new_edges)
